# baseline (speedup 1.0000x reference)
import functools

import jax
import jax.numpy as jnp
from jax.experimental import pallas as pl
from jax.experimental.pallas import tpu as pltpu
from jax.experimental.pallas import tpu_sc as plsc

_TEMP = 0.2
_RB = 8

_QSC = 4096
_CH = 2048
_QTAIL = 32
_SC_NW = 32
_SC_RPW = 8
_SC_L = 16


def _seg_kernel(fea_ref, pred_ref, sums_ref, cnt_ref, *, num_classes, hw):
    b = pl.program_id(0)
    p = pred_ref[0]
    best_v = p[0:1, :]
    best_i = jnp.zeros_like(best_v)
    for c in range(1, num_classes):
        v = p[c : c + 1, :]
        take = v > best_v
        best_v = jnp.where(take, v, best_v)
        best_i = jnp.where(take, jnp.float32(c), best_i)
    iota8 = jax.lax.broadcasted_iota(jnp.int32, (8, hw), 0).astype(jnp.float32)
    onehot = (iota8 == best_i).astype(jnp.float32)
    f = fea_ref[0]
    part = jax.lax.dot_general(
        f, onehot, (((1,), (1,)), ((), ())), preferred_element_type=jnp.float32
    )
    ones = jnp.ones((1, hw), jnp.float32)
    cnt = jax.lax.dot_general(
        ones, onehot, (((1,), (1,)), ((), ())), preferred_element_type=jnp.float32
    )

    @pl.when(b == 0)
    def _():
        sums_ref[...] = part
        cnt_ref[...] = cnt

    @pl.when(b > 0)
    def _():
        sums_ref[...] += part
        cnt_ref[...] += cnt


def _contrast_kernel(q_ref, k_ref, s_ref, *, num_classes):
    blk = q_ref[...]
    s = jnp.sum(blk, axis=0)
    for c in range(num_classes):
        x = blk[c]
        k = k_ref[:, c : c + 1]
        e = jnp.exp(x * k) + jnp.exp((s - x) * k)
        s_ref[:, c : c + 1] = jnp.sum(e, axis=1, keepdims=True)


def _sc_contrast(q_hbm, k_hbm, out_hbm, xbuf, kbuf, obuf, *, num_classes, q_start):
    wid = jax.lax.axis_index("s") * 2 + jax.lax.axis_index("c")
    base_r = wid * _SC_RPW
    pltpu.sync_copy(k_hbm.at[pl.ds(base_r, _SC_RPW)], kbuf)
    krows = [kbuf[r, :] for r in range(_SC_RPW)]
    for r in range(_SC_RPW):
        for c in range(num_classes):
            obuf[r, c] = jnp.zeros((_SC_L,), jnp.float32)

    def chunk_body(ch, carry):
        q0 = q_start + ch * _CH
        pltpu.sync_copy(
            q_hbm.at[:, pl.ds(base_r, _SC_RPW), pl.ds(q0, _CH)], xbuf
        )

        def qbody(qi, carry2):
            off = qi * _SC_L
            for r in range(_SC_RPW):
                xs = [xbuf[c2, r, pl.ds(off, _SC_L)] for c2 in range(num_classes)]
                s = xs[0]
                for c2 in range(1, num_classes):
                    s = s + xs[c2]
                for c2 in range(num_classes):
                    krc = krows[r][c2]
                    e = jnp.exp(xs[c2] * krc) + jnp.exp((s - xs[c2]) * krc)
                    plsc.addupdate(obuf.at[r, c2], e)
            return carry2

        jax.lax.fori_loop(0, _CH // _SC_L, qbody, 0)
        return carry

    jax.lax.fori_loop(0, _QSC // _CH, chunk_body, 0)
    pltpu.sync_copy(obuf, out_hbm.at[pl.ds(base_r, _SC_RPW)])


def _mask_kernel(
    plab_ref, s_ref, ssc_ref, qt_ref, k_ref, q0_ref, cnt_ref, th_ref, out_ref,
    *, num_classes, in_planes
):
    l0 = k_ref[...] * q0_ref[...]
    qt = qt_ref[...]
    st = jnp.sum(qt, axis=0)
    pmap = plab_ref[...]
    best_v = pmap[:, 0, :]
    best_i = jnp.zeros_like(best_v)
    for c in range(1, num_classes):
        v = pmap[:, c, :]
        take = v > best_v
        best_v = jnp.where(take, v, best_v)
        best_i = jnp.where(take, jnp.float32(c), best_i)
    out = best_i
    for c in range(num_classes):
        xt = qt[c]
        kc = k_ref[:, c : c + 1]
        et = jnp.exp(xt * kc) + jnp.exp((st - xt) * kc)
        s_c = (
            s_ref[:, c : c + 1]
            + jnp.sum(ssc_ref[:, c * _SC_L : (c + 1) * _SC_L], axis=1, keepdims=True)
            + jnp.sum(et, axis=1, keepdims=True)
        )
        loss_c = (jnp.sum(jnp.log(s_c)) - jnp.sum(l0[:, c])) / jnp.float32(in_planes)
        drop = jnp.logical_or(cnt_ref[c] <= 0.0, loss_c > th_ref[c])
        out = jnp.where(
            jnp.logical_and(drop, best_i == jnp.float32(c)), jnp.float32(-1.0), out
        )
    out_ref[...] = out


@jax.jit
def kernel(fea, pred, contrast_loss_input, pesudo_label, queues):
    bsz, in_planes, hgt, wid = fea.shape
    num_classes = pred.shape[1]
    queue_len = queues.shape[2]
    hw = hgt * wid

    fea3 = fea.reshape(bsz, in_planes, hw)
    pred3 = pred.reshape(bsz, num_classes, hw)
    plab3 = pesudo_label.reshape(bsz, num_classes, hw)

    sums8, cnt8 = pl.pallas_call(
        functools.partial(_seg_kernel, num_classes=num_classes, hw=hw),
        grid=(bsz,),
        in_specs=[
            pl.BlockSpec((1, in_planes, hw), lambda b: (b, 0, 0)),
            pl.BlockSpec((1, num_classes, hw), lambda b: (b, 0, 0)),
        ],
        out_specs=[
            pl.BlockSpec((in_planes, 8), lambda b: (0, 0)),
            pl.BlockSpec((1, 8), lambda b: (0, 0)),
        ],
        out_shape=[
            jax.ShapeDtypeStruct((in_planes, 8), jnp.float32),
            jax.ShapeDtypeStruct((1, 8), jnp.float32),
        ],
    )(fea3, pred3)

    sums = sums8[:, :num_classes]
    cnt = cnt8[0, :num_classes]
    means = sums / jnp.where(cnt > 0, cnt, 1.0)[None, :]
    norm = jnp.sqrt(jnp.sum(means * means, axis=0, keepdims=True))
    keys_t = means / jnp.maximum(norm, 1e-12)
    keys_scaled = keys_t * jnp.float32(1.0 / _TEMP)

    q_tc = queue_len - _QSC - _QTAIL

    keys16 = jnp.zeros((in_planes, _SC_L), jnp.float32).at[:, :num_classes].set(
        keys_scaled
    )
    sc_mesh = plsc.VectorSubcoreMesh(
        core_axis_name="c", subcore_axis_name="s", num_cores=2, num_subcores=16
    )
    s_sc = pl.kernel(
        functools.partial(_sc_contrast, num_classes=num_classes, q_start=q_tc),
        out_type=jax.ShapeDtypeStruct((in_planes, num_classes, _SC_L), jnp.float32),
        mesh=sc_mesh,
        scratch_types=[
            pltpu.VMEM((num_classes, _SC_RPW, _CH), jnp.float32),
            pltpu.VMEM((_SC_RPW, _SC_L), jnp.float32),
            pltpu.VMEM((_SC_RPW, num_classes, _SC_L), jnp.float32),
        ],
        cost_estimate=pl.CostEstimate(
            flops=8 * num_classes * in_planes * _QSC,
            transcendentals=2 * num_classes * in_planes * _QSC,
            bytes_accessed=4 * num_classes * in_planes * _QSC,
        ),
    )(queues, keys16)
    s_sc2 = s_sc.reshape(in_planes, num_classes * _SC_L)

    nr = in_planes // _RB
    s_tot = pl.pallas_call(
        functools.partial(_contrast_kernel, num_classes=num_classes),
        grid=(nr,),
        in_specs=[
            pl.BlockSpec((num_classes, _RB, q_tc), lambda r: (0, r, 0)),
            pl.BlockSpec((_RB, num_classes), lambda r: (r, 0)),
        ],
        out_specs=pl.BlockSpec((_RB, num_classes), lambda r: (r, 0)),
        out_shape=jax.ShapeDtypeStruct((in_planes, num_classes), jnp.float32),
    )(queues, keys_scaled)

    q0_t = queues[:, :, 0].T
    q_tail = queues[:, :, q_tc + _QSC :]

    out = pl.pallas_call(
        functools.partial(
            _mask_kernel, num_classes=num_classes, in_planes=in_planes
        ),
        grid=(1,),
        in_specs=[
            pl.BlockSpec((bsz, num_classes, hw), lambda i: (0, 0, 0)),
            pl.BlockSpec((in_planes, num_classes), lambda i: (0, 0)),
            pl.BlockSpec((in_planes, num_classes * _SC_L), lambda i: (0, 0)),
            pl.BlockSpec((num_classes, in_planes, _QTAIL), lambda i: (0, 0, 0)),
            pl.BlockSpec((in_planes, num_classes), lambda i: (0, 0)),
            pl.BlockSpec((in_planes, num_classes), lambda i: (0, 0)),
            pl.BlockSpec(memory_space=pltpu.SMEM),
            pl.BlockSpec(memory_space=pltpu.SMEM),
        ],
        out_specs=pl.BlockSpec((bsz, hw), lambda i: (0, 0)),
        out_shape=jax.ShapeDtypeStruct((bsz, hw), jnp.float32),
    )(plab3, s_tot, s_sc2, q_tail, keys_scaled, q0_t, cnt, contrast_loss_input)

    return out.reshape(bsz, hgt, wid)


# speedup vs baseline: 1.0986x; 1.0986x over previous
import functools

import jax
import jax.numpy as jnp
from jax.experimental import pallas as pl
from jax.experimental.pallas import tpu as pltpu
from jax.experimental.pallas import tpu_sc as plsc

_TEMP = 0.2
_QB = 1984

_QSC = 4096
_QTAIL = 32
_SC_NW = 32
_SC_CH = 64
_SC_L = 16


def _seg_kernel(fea_ref, pred_ref, sums_ref, cnt_ref, *, num_classes, hw):
    b = pl.program_id(0)
    p = pred_ref[0]
    best_v = p[0:1, :]
    best_i = jnp.zeros_like(best_v)
    for c in range(1, num_classes):
        v = p[c : c + 1, :]
        take = v > best_v
        best_v = jnp.where(take, v, best_v)
        best_i = jnp.where(take, jnp.float32(c), best_i)
    iota8 = jax.lax.broadcasted_iota(jnp.int32, (8, hw), 0).astype(jnp.float32)
    onehot = (iota8 == best_i).astype(jnp.float32)
    f = fea_ref[0]
    part = jax.lax.dot_general(
        onehot, f, (((1,), (0,)), ((), ())), preferred_element_type=jnp.float32
    )
    ones = jnp.ones((1, hw), jnp.float32)
    cnt = jax.lax.dot_general(
        ones, onehot, (((1,), (1,)), ((), ())), preferred_element_type=jnp.float32
    )

    @pl.when(b == 0)
    def _():
        sums_ref[...] = part
        cnt_ref[...] = cnt

    @pl.when(b > 0)
    def _():
        sums_ref[...] += part
        cnt_ref[...] += cnt


def _contrast_kernel(q_ref, k_ref, s_ref, *, num_classes):
    j = pl.program_id(0)

    @pl.when(j == 0)
    def _():
        s_ref[...] = jnp.zeros_like(s_ref)

    blk = q_ref[...]
    s = jnp.sum(blk, axis=0)
    for c in range(num_classes):
        x = blk[c]
        k = k_ref[c : c + 1, :]
        e = jnp.exp(x * k) + jnp.exp((s - x) * k)
        s_ref[c : c + 1, :] += jnp.sum(e, axis=0, keepdims=True)


def _sc_contrast(q_hbm, k_hbm, out_hbm, xbuf, kbuf, obuf, *, num_classes, q_start):
    wid = jax.lax.axis_index("s") * 2 + jax.lax.axis_index("c")
    qpw = _QSC // _SC_NW
    q0w = q_start + wid * qpw
    pltpu.sync_copy(k_hbm, kbuf)
    n16 = kbuf.shape[1] // _SC_L
    for c in range(num_classes):
        for g in range(n16):
            obuf[c, pl.ds(g * _SC_L, _SC_L)] = jnp.zeros((_SC_L,), jnp.float32)

    def chunk_body(ch, carry):
        pltpu.sync_copy(
            q_hbm.at[:, pl.ds(q0w + ch * _SC_CH, _SC_CH), :], xbuf
        )

        def qbody(qi, carry2):
            for g in range(n16):
                off = g * _SC_L
                xs = [
                    xbuf[c2, qi, pl.ds(off, _SC_L)] for c2 in range(num_classes)
                ]
                s = xs[0]
                for c2 in range(1, num_classes):
                    s = s + xs[c2]
                for c2 in range(num_classes):
                    kv = kbuf[c2, pl.ds(off, _SC_L)]
                    e = jnp.exp(xs[c2] * kv) + jnp.exp((s - xs[c2]) * kv)
                    plsc.addupdate(obuf.at[c2, pl.ds(off, _SC_L)], e)
            return carry2

        jax.lax.fori_loop(0, _SC_CH, qbody, 0)
        return carry

    jax.lax.fori_loop(0, qpw // _SC_CH, chunk_body, 0)
    pltpu.sync_copy(obuf, out_hbm.at[wid])


def _mask_kernel(
    plab_ref, s_ref, ssc_ref, qt_ref, k_ref, q0_ref, cnt_ref, th_ref, out_ref,
    *, num_classes, in_planes
):
    l0 = k_ref[...] * q0_ref[...]
    s_all = s_ref[...] + jnp.sum(ssc_ref[...], axis=0)
    qt = qt_ref[...]
    st = jnp.sum(qt, axis=0)
    pmap = plab_ref[...]
    best_v = pmap[:, 0, :]
    best_i = jnp.zeros_like(best_v)
    for c in range(1, num_classes):
        v = pmap[:, c, :]
        take = v > best_v
        best_v = jnp.where(take, v, best_v)
        best_i = jnp.where(take, jnp.float32(c), best_i)
    out = best_i
    for c in range(num_classes):
        xt = qt[c]
        kc = k_ref[c : c + 1, :]
        et = jnp.exp(xt * kc) + jnp.exp((st - xt) * kc)
        s_c = s_all[c : c + 1, :] + jnp.sum(et, axis=0, keepdims=True)
        loss_c = (jnp.sum(jnp.log(s_c)) - jnp.sum(l0[c : c + 1, :])) / jnp.float32(
            in_planes
        )
        drop = jnp.logical_or(cnt_ref[c] <= 0.0, loss_c > th_ref[c])
        out = jnp.where(
            jnp.logical_and(drop, best_i == jnp.float32(c)), jnp.float32(-1.0), out
        )
    out_ref[...] = out


@jax.jit
def kernel(fea, pred, contrast_loss_input, pesudo_label, queues):
    bsz, in_planes, hgt, wid = fea.shape
    num_classes = pred.shape[1]
    queue_len = queues.shape[2]
    hw = hgt * wid

    fea_r = jnp.transpose(fea, (0, 2, 3, 1)).reshape(bsz, hw, in_planes)
    q_t = jnp.transpose(queues, (0, 2, 1))
    pred3 = pred.reshape(bsz, num_classes, hw)
    plab3 = pesudo_label.reshape(bsz, num_classes, hw)

    sums8, cnt8 = pl.pallas_call(
        functools.partial(_seg_kernel, num_classes=num_classes, hw=hw),
        grid=(bsz,),
        in_specs=[
            pl.BlockSpec((1, hw, in_planes), lambda b: (b, 0, 0)),
            pl.BlockSpec((1, num_classes, hw), lambda b: (b, 0, 0)),
        ],
        out_specs=[
            pl.BlockSpec((8, in_planes), lambda b: (0, 0)),
            pl.BlockSpec((1, 8), lambda b: (0, 0)),
        ],
        out_shape=[
            jax.ShapeDtypeStruct((8, in_planes), jnp.float32),
            jax.ShapeDtypeStruct((1, 8), jnp.float32),
        ],
    )(fea_r, pred3)

    sums = sums8[:num_classes]
    cnt = cnt8[0, :num_classes]
    means = sums / jnp.where(cnt > 0, cnt, 1.0)[:, None]
    norm = jnp.sqrt(jnp.sum(means * means, axis=1, keepdims=True))
    keys = means / jnp.maximum(norm, 1e-12)
    keys_scaled = keys * jnp.float32(1.0 / _TEMP)

    q_tc = queue_len - _QSC - _QTAIL

    sc_mesh = plsc.VectorSubcoreMesh(
        core_axis_name="c", subcore_axis_name="s", num_cores=2, num_subcores=16
    )
    s_sc = pl.kernel(
        functools.partial(_sc_contrast, num_classes=num_classes, q_start=q_tc),
        out_type=jax.ShapeDtypeStruct((_SC_NW, num_classes, in_planes), jnp.float32),
        mesh=sc_mesh,
        scratch_types=[
            pltpu.VMEM((num_classes, _SC_CH, in_planes), jnp.float32),
            pltpu.VMEM((num_classes, in_planes), jnp.float32),
            pltpu.VMEM((num_classes, in_planes), jnp.float32),
        ],
        compiler_params=pltpu.CompilerParams(use_tc_tiling_on_sc=True),
        cost_estimate=pl.CostEstimate(
            flops=8 * num_classes * in_planes * _QSC,
            transcendentals=2 * num_classes * in_planes * _QSC,
            bytes_accessed=4 * num_classes * in_planes * _QSC,
        ),
    )(q_t, keys_scaled)

    nq = q_tc // _QB
    s_tot = pl.pallas_call(
        functools.partial(_contrast_kernel, num_classes=num_classes),
        grid=(nq,),
        in_specs=[
            pl.BlockSpec((num_classes, _QB, in_planes), lambda j: (0, j, 0)),
            pl.BlockSpec((num_classes, in_planes), lambda j: (0, 0)),
        ],
        out_specs=pl.BlockSpec((num_classes, in_planes), lambda j: (0, 0)),
        out_shape=jax.ShapeDtypeStruct((num_classes, in_planes), jnp.float32),
    )(q_t, keys_scaled)

    q0_t = queues[:, :, 0]
    q_tail = jnp.transpose(
        queues[:, :, q_tc + _QSC :], (0, 2, 1)
    )

    out = pl.pallas_call(
        functools.partial(
            _mask_kernel, num_classes=num_classes, in_planes=in_planes
        ),
        grid=(1,),
        in_specs=[
            pl.BlockSpec((bsz, num_classes, hw), lambda i: (0, 0, 0)),
            pl.BlockSpec((num_classes, in_planes), lambda i: (0, 0)),
            pl.BlockSpec((_SC_NW, num_classes, in_planes), lambda i: (0, 0, 0)),
            pl.BlockSpec((num_classes, _QTAIL, in_planes), lambda i: (0, 0, 0)),
            pl.BlockSpec((num_classes, in_planes), lambda i: (0, 0)),
            pl.BlockSpec((num_classes, in_planes), lambda i: (0, 0)),
            pl.BlockSpec(memory_space=pltpu.SMEM),
            pl.BlockSpec(memory_space=pltpu.SMEM),
        ],
        out_specs=pl.BlockSpec((bsz, hw), lambda i: (0, 0)),
        out_shape=jax.ShapeDtypeStruct((bsz, hw), jnp.float32),
    )(plab3, s_tot, s_sc, q_tail, keys_scaled, q0_t, cnt, contrast_loss_input)

    return out.reshape(bsz, hgt, wid)


# speedup vs baseline: 2.2651x; 2.0619x over previous
import functools

import jax
import jax.numpy as jnp
from jax.experimental import pallas as pl
from jax.experimental.pallas import tpu as pltpu
from jax.experimental.pallas import tpu_sc as plsc

_TEMP = 0.2
_QB = 2368

_QSC = 1024
_QTAIL = 32
_SC_NW = 32
_SC_CH = 32
_SC_L = 16


def _seg_kernel(fea_ref, pred_ref, sums_ref, cnt_ref, *, num_classes, hw):
    b = pl.program_id(0)
    p = pred_ref[0]
    best_v = p[0:1, :]
    best_i = jnp.zeros_like(best_v)
    for c in range(1, num_classes):
        v = p[c : c + 1, :]
        take = v > best_v
        best_v = jnp.where(take, v, best_v)
        best_i = jnp.where(take, jnp.float32(c), best_i)
    iota8 = jax.lax.broadcasted_iota(jnp.int32, (8, hw), 0).astype(jnp.float32)
    onehot = (iota8 == best_i).astype(jnp.float32)
    f = fea_ref[0]
    part = jax.lax.dot_general(
        onehot, f, (((1,), (0,)), ((), ())), preferred_element_type=jnp.float32
    )
    ones = jnp.ones((1, hw), jnp.float32)
    cnt = jax.lax.dot_general(
        ones, onehot, (((1,), (1,)), ((), ())), preferred_element_type=jnp.float32
    )

    @pl.when(b == 0)
    def _():
        sums_ref[...] = part
        cnt_ref[...] = cnt

    @pl.when(b > 0)
    def _():
        sums_ref[...] += part
        cnt_ref[...] += cnt


def _contrast_kernel(q_ref, k_ref, s_ref, *, num_classes):
    j = pl.program_id(0)

    @pl.when(j == 0)
    def _():
        s_ref[...] = jnp.zeros_like(s_ref)

    blk = q_ref[...]
    s = jnp.sum(blk, axis=0)
    for c in range(num_classes):
        x = blk[c]
        k = k_ref[c : c + 1, :]
        e = jnp.exp(x * k) + jnp.exp((s - x) * k)
        s_ref[c : c + 1, :] += jnp.sum(e, axis=0, keepdims=True)


def _sc_contrast(q_hbm, k_hbm, out_hbm, xbuf, kbuf, obuf, *, num_classes, q_start):
    wid = jax.lax.axis_index("s") * 2 + jax.lax.axis_index("c")
    qpw = _QSC // _SC_NW
    q0w = q_start + wid * qpw
    pltpu.sync_copy(k_hbm, kbuf)
    n16 = kbuf.shape[1] // _SC_L
    for c in range(num_classes):
        for g in range(n16):
            obuf[c, pl.ds(g * _SC_L, _SC_L)] = jnp.zeros((_SC_L,), jnp.float32)

    def chunk_body(ch, carry):
        pltpu.sync_copy(
            q_hbm.at[:, pl.ds(q0w + ch * _SC_CH, _SC_CH), :], xbuf
        )

        def qbody(qi, carry2):
            for g in range(n16):
                off = g * _SC_L
                xs = [
                    xbuf[c2, qi, pl.ds(off, _SC_L)] for c2 in range(num_classes)
                ]
                s = xs[0]
                for c2 in range(1, num_classes):
                    s = s + xs[c2]
                for c2 in range(num_classes):
                    kv = kbuf[c2, pl.ds(off, _SC_L)]
                    e = jnp.exp(xs[c2] * kv) + jnp.exp((s - xs[c2]) * kv)
                    plsc.addupdate(obuf.at[c2, pl.ds(off, _SC_L)], e)
            return carry2

        jax.lax.fori_loop(0, _SC_CH, qbody, 0)
        return carry

    jax.lax.fori_loop(0, qpw // _SC_CH, chunk_body, 0)
    pltpu.sync_copy(obuf, out_hbm.at[wid])


def _mask_kernel(
    plab_ref, s_ref, ssc_ref, qt_ref, k_ref, q0_ref, cnt_ref, th_ref, out_ref,
    *, num_classes, in_planes
):
    l0 = k_ref[...] * q0_ref[...]
    s_all = s_ref[...] + jnp.sum(ssc_ref[...], axis=0)
    qt = qt_ref[...]
    st = jnp.sum(qt, axis=0)
    pmap = plab_ref[...]
    best_v = pmap[:, 0, :]
    best_i = jnp.zeros_like(best_v)
    for c in range(1, num_classes):
        v = pmap[:, c, :]
        take = v > best_v
        best_v = jnp.where(take, v, best_v)
        best_i = jnp.where(take, jnp.float32(c), best_i)
    out = best_i
    for c in range(num_classes):
        xt = qt[c]
        kc = k_ref[c : c + 1, :]
        et = jnp.exp(xt * kc) + jnp.exp((st - xt) * kc)
        s_c = s_all[c : c + 1, :] + jnp.sum(et, axis=0, keepdims=True)
        loss_c = (jnp.sum(jnp.log(s_c)) - jnp.sum(l0[c : c + 1, :])) / jnp.float32(
            in_planes
        )
        drop = jnp.logical_or(cnt_ref[c] <= 0.0, loss_c > th_ref[c])
        out = jnp.where(
            jnp.logical_and(drop, best_i == jnp.float32(c)), jnp.float32(-1.0), out
        )
    out_ref[...] = out


@jax.jit
def kernel(fea, pred, contrast_loss_input, pesudo_label, queues):
    bsz, in_planes, hgt, wid = fea.shape
    num_classes = pred.shape[1]
    queue_len = queues.shape[2]
    hw = hgt * wid

    fea_r = jnp.transpose(fea, (0, 2, 3, 1)).reshape(bsz, hw, in_planes)
    q_t = jnp.transpose(queues, (0, 2, 1))
    pred3 = pred.reshape(bsz, num_classes, hw)
    plab3 = pesudo_label.reshape(bsz, num_classes, hw)

    sums8, cnt8 = pl.pallas_call(
        functools.partial(_seg_kernel, num_classes=num_classes, hw=hw),
        grid=(bsz,),
        in_specs=[
            pl.BlockSpec((1, hw, in_planes), lambda b: (b, 0, 0)),
            pl.BlockSpec((1, num_classes, hw), lambda b: (b, 0, 0)),
        ],
        out_specs=[
            pl.BlockSpec((8, in_planes), lambda b: (0, 0)),
            pl.BlockSpec((1, 8), lambda b: (0, 0)),
        ],
        out_shape=[
            jax.ShapeDtypeStruct((8, in_planes), jnp.float32),
            jax.ShapeDtypeStruct((1, 8), jnp.float32),
        ],
    )(fea_r, pred3)

    sums = sums8[:num_classes]
    cnt = cnt8[0, :num_classes]
    means = sums / jnp.where(cnt > 0, cnt, 1.0)[:, None]
    norm = jnp.sqrt(jnp.sum(means * means, axis=1, keepdims=True))
    keys = means / jnp.maximum(norm, 1e-12)
    keys_scaled = keys * jnp.float32(1.0 / _TEMP)

    q_tc = queue_len - _QSC - _QTAIL

    sc_mesh = plsc.VectorSubcoreMesh(
        core_axis_name="c", subcore_axis_name="s", num_cores=2, num_subcores=16
    )
    s_sc = pl.kernel(
        functools.partial(_sc_contrast, num_classes=num_classes, q_start=q_tc),
        out_type=jax.ShapeDtypeStruct((_SC_NW, num_classes, in_planes), jnp.float32),
        mesh=sc_mesh,
        scratch_types=[
            pltpu.VMEM((num_classes, _SC_CH, in_planes), jnp.float32),
            pltpu.VMEM((num_classes, in_planes), jnp.float32),
            pltpu.VMEM((num_classes, in_planes), jnp.float32),
        ],
        compiler_params=pltpu.CompilerParams(use_tc_tiling_on_sc=True),
        cost_estimate=pl.CostEstimate(
            flops=8 * num_classes * in_planes * _QSC,
            transcendentals=2 * num_classes * in_planes * _QSC,
            bytes_accessed=4 * num_classes * in_planes * _QSC,
        ),
    )(q_t, keys_scaled)

    nq = q_tc // _QB
    s_tot = pl.pallas_call(
        functools.partial(_contrast_kernel, num_classes=num_classes),
        grid=(nq,),
        in_specs=[
            pl.BlockSpec((num_classes, _QB, in_planes), lambda j: (0, j, 0)),
            pl.BlockSpec((num_classes, in_planes), lambda j: (0, 0)),
        ],
        out_specs=pl.BlockSpec((num_classes, in_planes), lambda j: (0, 0)),
        out_shape=jax.ShapeDtypeStruct((num_classes, in_planes), jnp.float32),
    )(q_t, keys_scaled)

    q0_t = queues[:, :, 0]
    q_tail = jnp.transpose(
        queues[:, :, q_tc + _QSC :], (0, 2, 1)
    )

    out = pl.pallas_call(
        functools.partial(
            _mask_kernel, num_classes=num_classes, in_planes=in_planes
        ),
        grid=(1,),
        in_specs=[
            pl.BlockSpec((bsz, num_classes, hw), lambda i: (0, 0, 0)),
            pl.BlockSpec((num_classes, in_planes), lambda i: (0, 0)),
            pl.BlockSpec((_SC_NW, num_classes, in_planes), lambda i: (0, 0, 0)),
            pl.BlockSpec((num_classes, _QTAIL, in_planes), lambda i: (0, 0, 0)),
            pl.BlockSpec((num_classes, in_planes), lambda i: (0, 0)),
            pl.BlockSpec((num_classes, in_planes), lambda i: (0, 0)),
            pl.BlockSpec(memory_space=pltpu.SMEM),
            pl.BlockSpec(memory_space=pltpu.SMEM),
        ],
        out_specs=pl.BlockSpec((bsz, hw), lambda i: (0, 0)),
        out_shape=jax.ShapeDtypeStruct((bsz, hw), jnp.float32),
    )(plab3, s_tot, s_sc, q_tail, keys_scaled, q0_t, cnt, contrast_loss_input)

    return out.reshape(bsz, hgt, wid)
